# Initial kernel scaffold; baseline (speedup 1.0000x reference)
import jax
import jax.numpy as jnp
from jax.experimental import pallas as pl


def kernel(x, edge_index, batch, W1l, W1r, a1, bg1, W2l, W2r, a2, bg2, W3l, W3r, a3, bg3, W4l, W4r, a4, bg4, P1root, P1rel, P1b, P2root, P2rel, P2b, P3root, P3rel, P3b, P4root, P4rel, P4b, L1w, L1b, L2w, L2b, L3w, L3b):
    raise NotImplementedError("write your pallas kernel here")


# speedup vs baseline: 1.3883x; 1.3883x over previous
import functools
import jax
import jax.numpy as jnp
from jax import lax
from jax.experimental import pallas as pl
from jax.experimental.pallas import tpu as pltpu

_B = 1024


def _ru(v, m):
    return (v + m - 1) // m * m


def _smem_blk(B):
    return pl.BlockSpec((1, 1, B), lambda i: (i, 0, 0), memory_space=pltpu.SMEM)


def _vmem_blk(B):
    return pl.BlockSpec((1, 1, B), lambda i: (i, 0, 0))


def _full(shape):
    nd = len(shape)
    return pl.BlockSpec(shape, lambda i: (0,) * nd)


def _full0(shape):
    nd = len(shape)
    return pl.BlockSpec(shape, lambda: (0,) * nd)


def _smem_full(shape):
    nd = len(shape)
    return pl.BlockSpec(shape, lambda i: (0,) * nd, memory_space=pltpu.SMEM)


def _smem_full0(shape):
    nd = len(shape)
    return pl.BlockSpec(shape, lambda: (0,) * nd, memory_space=pltpu.SMEM)


def _mm(a, w):
    m, kd = a.shape
    _, nn = w.shape

    def body(ar, wr, orf):
        orf[...] = jnp.dot(ar[...], wr[...], preferred_element_type=jnp.float32)

    return pl.pallas_call(
        body,
        out_shape=jax.ShapeDtypeStruct((m, nn), jnp.float32),
    )(a, w)


def _edge_logits(srcb, dstb, xl, xr, att_row):
    EB, _, B = srcb.shape
    NP, H = xl.shape
    G = 8

    def body(sr, dr, xlr, xrr, ar, er):
        a = ar[...]

        def lp(jb, _):
            j0 = jb * G
            rl = [xlr[pl.ds(sr[0, 0, j0 + u], 1), :] for u in range(G)]
            rr2 = [xrr[pl.ds(dr[0, 0, j0 + u], 1), :] for u in range(G)]
            v = jnp.concatenate(rl, axis=0) + jnp.concatenate(rr2, axis=0)
            h = jnp.where(v >= 0, v, 0.2 * v) * a
            er[0, pl.ds(j0, G), :] = jnp.sum(h, axis=1, keepdims=True)
            return 0

        lax.fori_loop(0, B // G, lp, 0)

    return pl.pallas_call(
        body,
        grid=(EB,),
        in_specs=[_smem_blk(B), _smem_blk(B), _full((NP, H)), _full((NP, H)),
                  _full((1, H))],
        out_specs=pl.BlockSpec((1, B, 1), lambda i: (i, 0, 0)),
        out_shape=jax.ShapeDtypeStruct((EB, B, 1), jnp.float32),
    )(srcb, dstb, xl, xr, att_row)


def _gmax(e):
    EB, B, _ = e.shape

    def body(er, mr):
        i = pl.program_id(0)
        bm = jnp.max(er[...])

        @pl.when(i == 0)
        def _():
            mr[0, 0] = bm

        @pl.when(i > 0)
        def _():
            mr[0, 0] = jnp.maximum(mr[0, 0], bm)

    return pl.pallas_call(
        body,
        grid=(EB,),
        in_specs=[pl.BlockSpec((1, B, 1), lambda i: (i, 0, 0))],
        out_specs=_smem_full((1, 1)),
        out_shape=jax.ShapeDtypeStruct((1, 1), jnp.float32),
    )(e)


def _expshift(e, m):
    EB, B, _ = e.shape

    def body(er, mr, wr):
        wr[...] = jnp.exp(er[...] - mr[0, 0])

    return pl.pallas_call(
        body,
        grid=(EB,),
        in_specs=[pl.BlockSpec((1, B, 1), lambda i: (i, 0, 0)),
                  _smem_full((1, 1))],
        out_specs=pl.BlockSpec((1, B, 1), lambda i: (i, 0, 0)),
        out_shape=jax.ShapeDtypeStruct((EB, B, 1), jnp.float32),
    )(e, m)


def _denom(wb, dstb, NP):
    EB, _, B = wb.shape

    def body(wr, dr, denr):
        i = pl.program_id(0)

        @pl.when(i == 0)
        def _():
            def z(jb, _):
                for u in range(8):
                    denr[0, jb * 8 + u] = 0.0
                return 0
            lax.fori_loop(0, NP // 8, z, 0)

        def lp(jb, _):
            for u in range(4):
                j = jb * 4 + u
                d = dr[0, 0, j]
                denr[0, d] = denr[0, d] + wr[0, 0, j]
            return 0

        lax.fori_loop(0, B // 4, lp, 0)

    return pl.pallas_call(
        body,
        grid=(EB,),
        in_specs=[_smem_blk(B), _smem_blk(B)],
        out_specs=_smem_full((1, NP)),
        out_shape=jax.ShapeDtypeStruct((1, NP), jnp.float32),
    )(wb, dstb)


def _aggregate(srcb, dstb, wb, den, xl, bg_row, n):
    EB, _, B = wb.shape
    NP, H = xl.shape
    G = 8

    def body(sr, dr, wr, denr, xlr, bgr, outr):
        i = pl.program_id(0)

        @pl.when(i == 0)
        def _():
            outr[...] = jnp.zeros((NP, H), jnp.float32)

        def lp(jb, _):
            j0 = jb * G
            rows = [xlr[pl.ds(sr[0, 0, j0 + u], 1), :] for u in range(G)]
            for u in range(G):
                j = j0 + u
                d = dr[0, 0, j]
                alpha = wr[0, 0, j] / (denr[0, d] + 1e-16)
                outr[pl.ds(d, 1), :] = outr[pl.ds(d, 1), :] + alpha * rows[u]
            return 0

        lax.fori_loop(0, B // G, lp, 0)

        @pl.when(i == EB - 1)
        def _():
            acc = outr[...] + bgr[...]
            acc = jnp.where(acc >= 0, acc, 0.0)
            rid = lax.broadcasted_iota(jnp.int32, (NP, H), 0)
            outr[...] = jnp.where(rid < n, acc, 0.0)

    return pl.pallas_call(
        body,
        grid=(EB,),
        in_specs=[_smem_blk(B), _smem_blk(B), _smem_blk(B), _smem_full((1, NP)),
                  _full((NP, H)), _full((1, H))],
        out_specs=_full((NP, H)),
        out_shape=jax.ShapeDtypeStruct((NP, H), jnp.float32),
    )(srcb, dstb, wb, den, xl, bg_row)


def _tagg(srcb, dstb, t_row):
    EB, _, B = srcb.shape
    _, NP = t_row.shape

    def body(sr, dr, tr, ar):
        i = pl.program_id(0)

        @pl.when(i == 0)
        def _():
            def z(jb, _):
                for u in range(8):
                    ar[0, jb * 8 + u] = 0.0
                return 0
            lax.fori_loop(0, NP // 8, z, 0)

        def lp(jb, _):
            for u in range(4):
                j = jb * 4 + u
                s = sr[0, 0, j]
                d = dr[0, 0, j]
                ar[0, d] = ar[0, d] + tr[0, s]
            return 0

        lax.fori_loop(0, B // 4, lp, 0)

    return pl.pallas_call(
        body,
        grid=(EB,),
        in_specs=[_smem_blk(B), _smem_blk(B), _smem_full((1, NP))],
        out_specs=_smem_full((1, NP)),
        out_shape=jax.ShapeDtypeStruct((1, NP), jnp.float32),
    )(srcb, dstb, t_row)


def _scorevec(agg_row, root_row, pb, n):
    _, NP = agg_row.shape

    def body(ar, rr, pr, scr, thr):
        sc = ar[...] + rr[...] + pr[0, 0]
        lane = lax.broadcasted_iota(jnp.int32, (1, NP), 1)
        sc = jnp.where(lane < n, sc, -1e30)
        scr[...] = sc
        thr[...] = jnp.tanh(sc)

    return pl.pallas_call(
        body,
        in_specs=[_full0((1, NP)), _full0((1, NP)), _smem_full0((1, 1))],
        out_specs=[_full0((1, NP)), _full0((1, NP))],
        out_shape=[jax.ShapeDtypeStruct((1, NP), jnp.float32),
                   jax.ShapeDtypeStruct((1, NP), jnp.float32)],
    )(agg_row, root_row, pb)


def _rank_select(score_col, score_row, k):
    NP = score_row.shape[1]
    R = 8

    def body(cr, rr, selr):
        i = pl.program_id(0)
        sc_i = cr[...]
        sc_j = rr[...]
        ig = i * R + lax.broadcasted_iota(jnp.int32, (R, 1), 0)
        jg = lax.broadcasted_iota(jnp.int32, (R, NP), 1)
        gt = (sc_j > sc_i).astype(jnp.int32)
        tie = ((sc_j == sc_i) & (jg < ig)).astype(jnp.int32)
        rank = jnp.sum(gt + tie, axis=1, keepdims=True)
        selr[...] = (rank < k).astype(jnp.int32)

    return pl.pallas_call(
        body,
        grid=(NP // R,),
        in_specs=[pl.BlockSpec((R, 1), lambda i: (i, 0)), _full((1, NP))],
        out_specs=pl.BlockSpec((R, 1), lambda i: (i, 0)),
        out_shape=jax.ShapeDtypeStruct((NP, 1), jnp.int32),
    )(score_col, score_row)


def _prefix_map(sel_col, sel_row, k):
    NP = sel_row.shape[1]
    R = 8

    def body(cr, rr, mr):
        i = pl.program_id(0)
        ig = i * R + lax.broadcasted_iota(jnp.int32, (R, 1), 0)
        jg = lax.broadcasted_iota(jnp.int32, (R, NP), 1)
        cnt = jnp.sum(rr[...] * (jg < ig).astype(jnp.int32), axis=1, keepdims=True)
        mr[...] = jnp.where(cr[...] > 0, cnt, k)

    return pl.pallas_call(
        body,
        grid=(NP // R,),
        in_specs=[pl.BlockSpec((R, 1), lambda i: (i, 0)), _full((1, NP))],
        out_specs=pl.BlockSpec((R, 1), lambda i: (i, 0)),
        out_shape=jax.ShapeDtypeStruct((NP, 1), jnp.int32),
    )(sel_col, sel_row)


def _permbuild(map_row, k, KP):
    NP = map_row.shape[1]

    def body(mr, prf):
        def lp(i, _):
            m = mr[0, i]
            prf[0, jnp.minimum(m, k)] = i
            return 0

        lax.fori_loop(0, NP, lp, 0)

    return pl.pallas_call(
        body,
        in_specs=[_smem_full0((1, NP))],
        out_specs=_smem_full0((1, KP)),
        out_shape=jax.ShapeDtypeStruct((1, KP), jnp.int32),
    )(map_row)


def _remap(srcb, dstb, map_row, k):
    EB, _, B = srcb.shape
    NP = map_row.shape[1]

    def body(sr, dr, mr, nsr, ndr):
        def lp(jb, _):
            for u in range(4):
                j = jb * 4 + u
                a = mr[0, sr[0, 0, j]]
                b = mr[0, dr[0, 0, j]]
                inv = (a == k) | (b == k)
                nsr[0, 0, j] = jnp.where(inv, k, a)
                ndr[0, 0, j] = jnp.where(inv, k, b)
            return 0

        lax.fori_loop(0, B // 4, lp, 0)

    return pl.pallas_call(
        body,
        grid=(EB,),
        in_specs=[_smem_blk(B), _smem_blk(B), _smem_full((1, NP))],
        out_specs=[_smem_blk(B), _smem_blk(B)],
        out_shape=[jax.ShapeDtypeStruct((EB, 1, B), jnp.int32),
                   jax.ShapeDtypeStruct((EB, 1, B), jnp.int32)],
    )(srcb, dstb, map_row)


def _pool(perm, th_row, x, k, NPn):
    NP, H = x.shape
    KP = perm.shape[1]

    def body(pr, thr, xr, xnr, ror):
        xnr[...] = jnp.zeros((NPn, H), jnp.float32)
        ror[...] = jnp.zeros((8, H), jnp.float32)

        def lp(j, carry):
            mx, sm = carry
            i = pr[0, j]
            r = xr[pl.ds(i, 1), :] * thr[0, i]
            xnr[pl.ds(j, 1), :] = r
            return jnp.maximum(mx, r), sm + r

        init = (jnp.full((1, H), -1e30, jnp.float32), jnp.zeros((1, H), jnp.float32))
        mx, sm = lax.fori_loop(0, k, lp, init)
        ror[pl.ds(0, 1), :] = mx
        ror[pl.ds(1, 1), :] = sm / k

    return pl.pallas_call(
        body,
        in_specs=[_smem_full0((1, KP)), _smem_full0((1, NP)), _full0((NP, H))],
        out_specs=[_full0((NPn, H)), _full0((8, H))],
        out_shape=[jax.ShapeDtypeStruct((NPn, H), jnp.float32),
                   jax.ShapeDtypeStruct((8, H), jnp.float32)],
    )(perm, th_row, x)


def _mlp(gs, L1w, L1b, L2w, L2b, L3wp, L3bp):
    def body(g1, g2, g3, g4, w1, b1, w2, b2, w3, b3, lgr, prr):
        g = g1[...] + g2[...] + g3[...] + g4[...]
        h1 = jnp.dot(g, w1[...], preferred_element_type=jnp.float32) + b1[...]
        h1 = jnp.where(h1 >= 0, h1, 0.0)
        h2 = jnp.dot(h1, w2[...], preferred_element_type=jnp.float32) + b2[...]
        h2 = jnp.where(h2 >= 0, h2, 0.0)
        lg = jnp.dot(h2, w3[...], preferred_element_type=jnp.float32) + b3[...]
        lane = lax.broadcasted_iota(jnp.int32, lg.shape, 1)
        valid = lane < 2
        lgm = jnp.where(valid, lg, -jnp.inf)
        z = lgm - jnp.max(lgm)
        ez = jnp.where(valid, jnp.exp(z), 0.0)
        prr[...] = ez / jnp.sum(ez)
        lgr[...] = lg

    n_in = 7
    return pl.pallas_call(
        body,
        in_specs=[_full0(a.shape) for a in gs] +
                 [_full0(L1w.shape), _full0(L1b.shape), _full0(L2w.shape),
                  _full0(L2b.shape), _full0(L3wp.shape), _full0(L3bp.shape)],
        out_specs=[_full0((1, 128)), _full0((1, 128))],
        out_shape=[jax.ShapeDtypeStruct((1, 128), jnp.float32),
                   jax.ShapeDtypeStruct((1, 128), jnp.float32)],
    )(*gs, L1w, L1b, L2w, L2b, L3wp, L3bp)


def _blocks(idx, fill, B):
    n = idx.shape[0]
    EB = _ru(n, B) // B
    pad = jnp.full((EB * B - n,), fill, jnp.int32)
    return jnp.concatenate([idx.astype(jnp.int32), pad]).reshape(EB, 1, B)


@jax.jit
def kernel(x, edge_index, batch, W1l, W1r, a1, bg1, W2l, W2r, a2, bg2, W3l, W3r,
           a3, bg3, W4l, W4r, a4, bg4, P1root, P1rel, P1b, P2root, P2rel, P2b,
           P3root, P3rel, P3b, P4root, P4rel, P4b, L1w, L1b, L2w, L2b, L3w, L3b):
    N0, DIN = x.shape
    E = edge_index.shape[1]
    H = W1l.shape[1]

    gat_w = [(W1l, W1r, a1, bg1), (W2l, W2r, a2, bg2), (W3l, W3r, a3, bg3),
             (W4l, W4r, a4, bg4)]
    pool_w = [(P1root, P1rel, P1b), (P2root, P2rel, P2b), (P3root, P3rel, P3b),
              (P4root, P4rel, P4b)]

    src = edge_index[0].astype(jnp.int32)
    dst = edge_index[1].astype(jnp.int32)
    n = N0
    NP = _ru(n + 1, 128)
    Xp = jnp.zeros((NP, DIN), jnp.float32).at[:n].set(x)

    readouts = []
    for li in range(4):
        Wl, Wr, att, bg = gat_w[li]
        Proot, Prel, Pb = pool_w[li]
        k = n // 2

        loops = jnp.arange(n, dtype=jnp.int32)
        s_full = jnp.concatenate([src, loops])
        d_full = jnp.concatenate([dst, loops])
        sb = _blocks(s_full, n, _B)
        db = _blocks(d_full, n, _B)

        xl = _mm(Xp, Wl)
        xr = _mm(Xp, Wr)
        att_row = att.reshape(1, H)

        e = _edge_logits(sb, db, xl, xr, att_row)
        m = _gmax(e)
        w = _expshift(e, m)
        EBn = e.shape[0]
        w13 = w.reshape(EBn, 1, _B)
        den = _denom(w13, db, NP)
        xg = _aggregate(sb, db, w13, den, xl, bg.reshape(1, H), n)

        W2c = jnp.zeros((H, 128), jnp.float32)
        W2c = W2c.at[:, 0].set(Prel[:, 0]).at[:, 1].set(Proot[:, 0])
        tc = _mm(xg, W2c)
        t_row = tc[:, 0:1].reshape(1, NP)
        root_row = tc[:, 1:2].reshape(1, NP)

        sbo = _blocks(src, n, _B)
        dbo = _blocks(dst, n, _B)
        agg_row = _tagg(sbo, dbo, t_row)
        score, th = _scorevec(agg_row, root_row, Pb.reshape(1, 1), n)

        score_col = score.reshape(NP, 1)
        sel = _rank_select(score_col, score, k)
        mapping = _prefix_map(sel, sel.reshape(1, NP), k)
        map_row = mapping.reshape(1, NP)

        KP = _ru(k + 1, 128)
        perm = _permbuild(map_row, k, KP)

        nsb, ndb = _remap(sbo, dbo, map_row, k)
        src = nsb.reshape(-1)[:E]
        dst = ndb.reshape(-1)[:E]

        NPn = _ru(k + 1, 128)
        Xp, ro = _pool(perm, th, xg, k, NPn)
        readouts.append(ro[0:2, :].reshape(1, 2 * H))
        n = k
        NP = NPn

    lgp, prp = _mlp(readouts, L1w, L1b.reshape(1, H),
                    L2w, L2b.reshape(1, H // 2),
                    jnp.pad(L3w, ((0, 0), (0, 126))),
                    jnp.pad(L3b.reshape(1, 2), ((0, 0), (0, 126))))
    return lgp[:, :2], prp[:, :2]


# speedup vs baseline: 1.7169x; 1.2367x over previous
import functools
import jax
import jax.numpy as jnp
from jax import lax
from jax.experimental import pallas as pl
from jax.experimental.pallas import tpu as pltpu

_B = 1024


def _ru(v, m):
    return (v + m - 1) // m * m


def _smem_blk(B):
    return pl.BlockSpec((1, 1, B), lambda i: (i, 0, 0), memory_space=pltpu.SMEM)


def _vmem_blk(B):
    return pl.BlockSpec((1, 1, B), lambda i: (i, 0, 0))


def _full(shape):
    nd = len(shape)
    return pl.BlockSpec(shape, lambda i: (0,) * nd)


def _full0(shape):
    nd = len(shape)
    return pl.BlockSpec(shape, lambda: (0,) * nd)


def _smem_full(shape):
    nd = len(shape)
    return pl.BlockSpec(shape, lambda i: (0,) * nd, memory_space=pltpu.SMEM)


def _smem_full0(shape):
    nd = len(shape)
    return pl.BlockSpec(shape, lambda: (0,) * nd, memory_space=pltpu.SMEM)


def _mm(a, w):
    m, kd = a.shape
    _, nn = w.shape

    def body(ar, wr, orf):
        orf[...] = jnp.dot(ar[...], wr[...], preferred_element_type=jnp.float32)

    return pl.pallas_call(
        body,
        out_shape=jax.ShapeDtypeStruct((m, nn), jnp.float32),
    )(a, w)


def _edge_logits(srcb, dstb, xl, xr, att_row):
    EB, _, B = srcb.shape
    NP, H = xl.shape
    G = 16

    def body(sr, dr, xlr, xrr, ar, er):
        a = ar[...]

        def lp(jb, _):
            j0 = jb * G
            rl = [xlr[pl.ds(sr[0, 0, j0 + u], 1), :] for u in range(G)]
            rr2 = [xrr[pl.ds(dr[0, 0, j0 + u], 1), :] for u in range(G)]
            v = jnp.concatenate(rl, axis=0) + jnp.concatenate(rr2, axis=0)
            h = jnp.where(v >= 0, v, 0.2 * v) * a
            er[0, pl.ds(j0, G), :] = jnp.sum(h, axis=1, keepdims=True)
            return 0

        lax.fori_loop(0, B // G, lp, 0)

    return pl.pallas_call(
        body,
        grid=(EB,),
        in_specs=[_smem_blk(B), _smem_blk(B), _full((NP, H)), _full((NP, H)),
                  _full((1, H))],
        out_specs=pl.BlockSpec((1, B, 1), lambda i: (i, 0, 0)),
        out_shape=jax.ShapeDtypeStruct((EB, B, 1), jnp.float32),
    )(srcb, dstb, xl, xr, att_row)


def _gmax(e):
    EB, B, _ = e.shape

    def body(er, mr):
        i = pl.program_id(0)
        bm = jnp.max(er[...])

        @pl.when(i == 0)
        def _():
            mr[0, 0] = bm

        @pl.when(i > 0)
        def _():
            mr[0, 0] = jnp.maximum(mr[0, 0], bm)

    return pl.pallas_call(
        body,
        grid=(EB,),
        in_specs=[pl.BlockSpec((1, B, 1), lambda i: (i, 0, 0))],
        out_specs=_smem_full((1, 1)),
        out_shape=jax.ShapeDtypeStruct((1, 1), jnp.float32),
    )(e)


def _expshift(e, m):
    EB, B, _ = e.shape

    def body(er, mr, wr):
        wr[...] = jnp.exp(er[...] - mr[0, 0])

    return pl.pallas_call(
        body,
        grid=(EB,),
        in_specs=[pl.BlockSpec((1, B, 1), lambda i: (i, 0, 0)),
                  _smem_full((1, 1))],
        out_specs=pl.BlockSpec((1, B, 1), lambda i: (i, 0, 0)),
        out_shape=jax.ShapeDtypeStruct((EB, B, 1), jnp.float32),
    )(e, m)


def _denom(wb, dstb, NP):
    EB, _, B = wb.shape

    def body(wr, dr, denr):
        i = pl.program_id(0)

        @pl.when(i == 0)
        def _():
            def z(jb, _):
                for u in range(8):
                    denr[0, jb * 8 + u] = 0.0
                return 0
            lax.fori_loop(0, NP // 8, z, 0)

        def lp(jb, _):
            for u in range(8):
                j = jb * 8 + u
                d = dr[0, 0, j]
                denr[0, d] = denr[0, d] + wr[0, 0, j]
            return 0

        lax.fori_loop(0, B // 8, lp, 0)

    return pl.pallas_call(
        body,
        grid=(EB,),
        in_specs=[_smem_blk(B), _smem_blk(B)],
        out_specs=_smem_full((1, NP)),
        out_shape=jax.ShapeDtypeStruct((1, NP), jnp.float32),
    )(wb, dstb)


def _aggregate(srcb, dstb, wb, den, xl, bg_row, n):
    EB, _, B = wb.shape
    NP, H = xl.shape
    G = 16

    def body(sr, dr, wr, denr, xlr, bgr, outr):
        i = pl.program_id(0)

        @pl.when(i == 0)
        def _():
            outr[...] = jnp.zeros((NP, H), jnp.float32)

        def lp(jb, _):
            j0 = jb * G
            rows = [xlr[pl.ds(sr[0, 0, j0 + u], 1), :] for u in range(G)]
            for u in range(G):
                j = j0 + u
                d = dr[0, 0, j]
                alpha = wr[0, 0, j] / (denr[0, d] + 1e-16)
                outr[pl.ds(d, 1), :] = outr[pl.ds(d, 1), :] + alpha * rows[u]
            return 0

        lax.fori_loop(0, B // G, lp, 0)

        @pl.when(i == EB - 1)
        def _():
            acc = outr[...] + bgr[...]
            acc = jnp.where(acc >= 0, acc, 0.0)
            rid = lax.broadcasted_iota(jnp.int32, (NP, H), 0)
            outr[...] = jnp.where(rid < n, acc, 0.0)

    return pl.pallas_call(
        body,
        grid=(EB,),
        in_specs=[_smem_blk(B), _smem_blk(B), _smem_blk(B), _smem_full((1, NP)),
                  _full((NP, H)), _full((1, H))],
        out_specs=_full((NP, H)),
        out_shape=jax.ShapeDtypeStruct((NP, H), jnp.float32),
    )(srcb, dstb, wb, den, xl, bg_row)


def _tagg(srcb, dstb, t_row):
    EB, _, B = srcb.shape
    _, NP = t_row.shape

    def body(sr, dr, tr, ar):
        i = pl.program_id(0)

        @pl.when(i == 0)
        def _():
            def z(jb, _):
                for u in range(8):
                    ar[0, jb * 8 + u] = 0.0
                return 0
            lax.fori_loop(0, NP // 8, z, 0)

        def lp(jb, _):
            for u in range(8):
                j = jb * 8 + u
                s = sr[0, 0, j]
                d = dr[0, 0, j]
                ar[0, d] = ar[0, d] + tr[0, s]
            return 0

        lax.fori_loop(0, B // 8, lp, 0)

    return pl.pallas_call(
        body,
        grid=(EB,),
        in_specs=[_smem_blk(B), _smem_blk(B), _smem_full((1, NP))],
        out_specs=_smem_full((1, NP)),
        out_shape=jax.ShapeDtypeStruct((1, NP), jnp.float32),
    )(srcb, dstb, t_row)


def _scorevec(agg_row, root_row, pb, n):
    _, NP = agg_row.shape

    def body(ar, rr, pr, scr, thr):
        sc = ar[...] + rr[...] + pr[0, 0]
        lane = lax.broadcasted_iota(jnp.int32, (1, NP), 1)
        sc = jnp.where(lane < n, sc, -1e30)
        scr[...] = sc
        thr[...] = jnp.tanh(sc)

    return pl.pallas_call(
        body,
        in_specs=[_full0((1, NP)), _full0((1, NP)), _smem_full0((1, 1))],
        out_specs=[_full0((1, NP)), _full0((1, NP))],
        out_shape=[jax.ShapeDtypeStruct((1, NP), jnp.float32),
                   jax.ShapeDtypeStruct((1, NP), jnp.float32)],
    )(agg_row, root_row, pb)


def _rank_select(score_col, score_row, k):
    NP = score_row.shape[1]
    R = 8

    def body(cr, rr, selr):
        i = pl.program_id(0)
        sc_i = cr[...]
        sc_j = rr[...]
        ig = i * R + lax.broadcasted_iota(jnp.int32, (R, 1), 0)
        jg = lax.broadcasted_iota(jnp.int32, (R, NP), 1)
        gt = (sc_j > sc_i).astype(jnp.int32)
        tie = ((sc_j == sc_i) & (jg < ig)).astype(jnp.int32)
        rank = jnp.sum(gt + tie, axis=1, keepdims=True)
        selr[...] = (rank < k).astype(jnp.int32)

    return pl.pallas_call(
        body,
        grid=(NP // R,),
        in_specs=[pl.BlockSpec((R, 1), lambda i: (i, 0)), _full((1, NP))],
        out_specs=pl.BlockSpec((R, 1), lambda i: (i, 0)),
        out_shape=jax.ShapeDtypeStruct((NP, 1), jnp.int32),
    )(score_col, score_row)


def _prefix_map(sel_col, sel_row, k):
    NP = sel_row.shape[1]
    R = 8

    def body(cr, rr, mr):
        i = pl.program_id(0)
        ig = i * R + lax.broadcasted_iota(jnp.int32, (R, 1), 0)
        jg = lax.broadcasted_iota(jnp.int32, (R, NP), 1)
        cnt = jnp.sum(rr[...] * (jg < ig).astype(jnp.int32), axis=1, keepdims=True)
        mr[...] = jnp.where(cr[...] > 0, cnt, k)

    return pl.pallas_call(
        body,
        grid=(NP // R,),
        in_specs=[pl.BlockSpec((R, 1), lambda i: (i, 0)), _full((1, NP))],
        out_specs=pl.BlockSpec((R, 1), lambda i: (i, 0)),
        out_shape=jax.ShapeDtypeStruct((NP, 1), jnp.int32),
    )(sel_col, sel_row)


def _permbuild(map_row, k, KP):
    NP = map_row.shape[1]

    def body(mr, prf):
        def lp(ib, _):
            for u in range(4):
                i = ib * 4 + u
                m = mr[0, i]
                prf[0, jnp.minimum(m, k)] = i
            return 0

        lax.fori_loop(0, NP // 4, lp, 0)

    return pl.pallas_call(
        body,
        in_specs=[_smem_full0((1, NP))],
        out_specs=_smem_full0((1, KP)),
        out_shape=jax.ShapeDtypeStruct((1, KP), jnp.int32),
    )(map_row)


def _remap(srcb, dstb, map_row, k):
    EB, _, B = srcb.shape
    NP = map_row.shape[1]

    def body(sr, dr, mr, nsr, ndr):
        def lp(jb, _):
            for u in range(8):
                j = jb * 8 + u
                a = mr[0, sr[0, 0, j]]
                b = mr[0, dr[0, 0, j]]
                inv = (a == k) | (b == k)
                nsr[0, 0, j] = jnp.where(inv, k, a)
                ndr[0, 0, j] = jnp.where(inv, k, b)
            return 0

        lax.fori_loop(0, B // 8, lp, 0)

    return pl.pallas_call(
        body,
        grid=(EB,),
        in_specs=[_smem_blk(B), _smem_blk(B), _smem_full((1, NP))],
        out_specs=[_smem_blk(B), _smem_blk(B)],
        out_shape=[jax.ShapeDtypeStruct((EB, 1, B), jnp.int32),
                   jax.ShapeDtypeStruct((EB, 1, B), jnp.int32)],
    )(srcb, dstb, map_row)


def _pool(perm, th_row, x, k, NPn):
    NP, H = x.shape
    KP = perm.shape[1]

    def body(pr, thr, xr, xnr, ror):
        xnr[...] = jnp.zeros((NPn, H), jnp.float32)
        ror[...] = jnp.zeros((8, H), jnp.float32)

        def lp(j, carry):
            mx, sm = carry
            i = pr[0, j]
            r = xr[pl.ds(i, 1), :] * thr[0, i]
            xnr[pl.ds(j, 1), :] = r
            return jnp.maximum(mx, r), sm + r

        init = (jnp.full((1, H), -1e30, jnp.float32), jnp.zeros((1, H), jnp.float32))
        mx, sm = lax.fori_loop(0, k, lp, init)
        ror[pl.ds(0, 1), :] = mx
        ror[pl.ds(1, 1), :] = sm / k

    return pl.pallas_call(
        body,
        in_specs=[_smem_full0((1, KP)), _smem_full0((1, NP)), _full0((NP, H))],
        out_specs=[_full0((NPn, H)), _full0((8, H))],
        out_shape=[jax.ShapeDtypeStruct((NPn, H), jnp.float32),
                   jax.ShapeDtypeStruct((8, H), jnp.float32)],
    )(perm, th_row, x)


def _mlp(gs, L1w, L1b, L2w, L2b, L3wp, L3bp):
    def body(g1, g2, g3, g4, w1, b1, w2, b2, w3, b3, lgr, prr):
        g = g1[...] + g2[...] + g3[...] + g4[...]
        h1 = jnp.dot(g, w1[...], preferred_element_type=jnp.float32) + b1[...]
        h1 = jnp.where(h1 >= 0, h1, 0.0)
        h2 = jnp.dot(h1, w2[...], preferred_element_type=jnp.float32) + b2[...]
        h2 = jnp.where(h2 >= 0, h2, 0.0)
        lg = jnp.dot(h2, w3[...], preferred_element_type=jnp.float32) + b3[...]
        lane = lax.broadcasted_iota(jnp.int32, lg.shape, 1)
        valid = lane < 2
        lgm = jnp.where(valid, lg, -jnp.inf)
        z = lgm - jnp.max(lgm)
        ez = jnp.where(valid, jnp.exp(z), 0.0)
        prr[...] = ez / jnp.sum(ez)
        lgr[...] = lg

    n_in = 7
    return pl.pallas_call(
        body,
        in_specs=[_full0(a.shape) for a in gs] +
                 [_full0(L1w.shape), _full0(L1b.shape), _full0(L2w.shape),
                  _full0(L2b.shape), _full0(L3wp.shape), _full0(L3bp.shape)],
        out_specs=[_full0((1, 128)), _full0((1, 128))],
        out_shape=[jax.ShapeDtypeStruct((1, 128), jnp.float32),
                   jax.ShapeDtypeStruct((1, 128), jnp.float32)],
    )(*gs, L1w, L1b, L2w, L2b, L3wp, L3bp)


def _blocks(idx, fill, B):
    n = idx.shape[0]
    EB = _ru(n, B) // B
    pad = jnp.full((EB * B - n,), fill, jnp.int32)
    return jnp.concatenate([idx.astype(jnp.int32), pad]).reshape(EB, 1, B)


@jax.jit
def kernel(x, edge_index, batch, W1l, W1r, a1, bg1, W2l, W2r, a2, bg2, W3l, W3r,
           a3, bg3, W4l, W4r, a4, bg4, P1root, P1rel, P1b, P2root, P2rel, P2b,
           P3root, P3rel, P3b, P4root, P4rel, P4b, L1w, L1b, L2w, L2b, L3w, L3b):
    N0, DIN = x.shape
    E = edge_index.shape[1]
    H = W1l.shape[1]

    gat_w = [(W1l, W1r, a1, bg1), (W2l, W2r, a2, bg2), (W3l, W3r, a3, bg3),
             (W4l, W4r, a4, bg4)]
    pool_w = [(P1root, P1rel, P1b), (P2root, P2rel, P2b), (P3root, P3rel, P3b),
              (P4root, P4rel, P4b)]

    src = edge_index[0].astype(jnp.int32)
    dst = edge_index[1].astype(jnp.int32)
    n = N0
    NP = _ru(n + 1, 128)
    Xp = jnp.zeros((NP, DIN), jnp.float32).at[:n].set(x)

    readouts = []
    for li in range(4):
        Wl, Wr, att, bg = gat_w[li]
        Proot, Prel, Pb = pool_w[li]
        k = n // 2

        loops = jnp.arange(n, dtype=jnp.int32)
        s_full = jnp.concatenate([src, loops])
        d_full = jnp.concatenate([dst, loops])
        sb = _blocks(s_full, n, _B)
        db = _blocks(d_full, n, _B)

        xl = _mm(Xp, Wl)
        xr = _mm(Xp, Wr)
        att_row = att.reshape(1, H)

        e = _edge_logits(sb, db, xl, xr, att_row)
        m = _gmax(e)
        w = _expshift(e, m)
        EBn = e.shape[0]
        w13 = w.reshape(EBn, 1, _B)
        den = _denom(w13, db, NP)
        xg = _aggregate(sb, db, w13, den, xl, bg.reshape(1, H), n)

        W2c = jnp.zeros((H, 128), jnp.float32)
        W2c = W2c.at[:, 0].set(Prel[:, 0]).at[:, 1].set(Proot[:, 0])
        tc = _mm(xg, W2c)
        t_row = tc[:, 0:1].reshape(1, NP)
        root_row = tc[:, 1:2].reshape(1, NP)

        sbo = _blocks(src, n, _B)
        dbo = _blocks(dst, n, _B)
        agg_row = _tagg(sbo, dbo, t_row)
        score, th = _scorevec(agg_row, root_row, Pb.reshape(1, 1), n)

        score_col = score.reshape(NP, 1)
        sel = _rank_select(score_col, score, k)
        mapping = _prefix_map(sel, sel.reshape(1, NP), k)
        map_row = mapping.reshape(1, NP)

        KP = _ru(k + 1, 128)
        perm = _permbuild(map_row, k, KP)

        nsb, ndb = _remap(sbo, dbo, map_row, k)
        src = nsb.reshape(-1)[:E]
        dst = ndb.reshape(-1)[:E]

        NPn = _ru(k + 1, 128)
        Xp, ro = _pool(perm, th, xg, k, NPn)
        readouts.append(ro[0:2, :].reshape(1, 2 * H))
        n = k
        NP = NPn

    lgp, prp = _mlp(readouts, L1w, L1b.reshape(1, H),
                    L2w, L2b.reshape(1, H // 2),
                    jnp.pad(L3w, ((0, 0), (0, 126))),
                    jnp.pad(L3b.reshape(1, 2), ((0, 0), (0, 126))))
    return lgp[:, :2], prp[:, :2]


# speedup vs baseline: 1.9577x; 1.1403x over previous
import functools
import jax
import jax.numpy as jnp
from jax import lax
from jax.experimental import pallas as pl
from jax.experimental.pallas import tpu as pltpu

_B = 1024


def _ru(v, m):
    return (v + m - 1) // m * m


def _smem_blk(B):
    return pl.BlockSpec((1, 1, B), lambda i: (i, 0, 0), memory_space=pltpu.SMEM)


def _vmem_blk(B):
    return pl.BlockSpec((1, 1, B), lambda i: (i, 0, 0))


def _full(shape):
    nd = len(shape)
    return pl.BlockSpec(shape, lambda i: (0,) * nd)


def _full0(shape):
    nd = len(shape)
    return pl.BlockSpec(shape, lambda: (0,) * nd)


def _smem_full(shape):
    nd = len(shape)
    return pl.BlockSpec(shape, lambda i: (0,) * nd, memory_space=pltpu.SMEM)


def _smem_full0(shape):
    nd = len(shape)
    return pl.BlockSpec(shape, lambda: (0,) * nd, memory_space=pltpu.SMEM)


def _mm(a, w):
    m, kd = a.shape
    _, nn = w.shape

    def body(ar, wr, orf):
        orf[...] = jnp.dot(ar[...], wr[...], preferred_element_type=jnp.float32)

    return pl.pallas_call(
        body,
        out_shape=jax.ShapeDtypeStruct((m, nn), jnp.float32),
    )(a, w)


def _edge_logits(srcb, dstb, xl, xr, att_row):
    EB, _, B = srcb.shape
    NP, H = xl.shape
    G = 32

    def body(sr, dr, xlr, xrr, ar, er):
        a = ar[...]

        def lp(jb, _):
            j0 = jb * G
            rl = [xlr[pl.ds(sr[0, 0, j0 + u], 1), :] for u in range(G)]
            rr2 = [xrr[pl.ds(dr[0, 0, j0 + u], 1), :] for u in range(G)]
            v = jnp.concatenate(rl, axis=0) + jnp.concatenate(rr2, axis=0)
            h = jnp.where(v >= 0, v, 0.2 * v) * a
            er[0, pl.ds(j0, G), :] = jnp.sum(h, axis=1, keepdims=True)
            return 0

        lax.fori_loop(0, B // G, lp, 0)

    return pl.pallas_call(
        body,
        grid=(EB,),
        in_specs=[_smem_blk(B), _smem_blk(B), _full((NP, H)), _full((NP, H)),
                  _full((1, H))],
        out_specs=pl.BlockSpec((1, B, 1), lambda i: (i, 0, 0)),
        out_shape=jax.ShapeDtypeStruct((EB, B, 1), jnp.float32),
    )(srcb, dstb, xl, xr, att_row)


def _gmax(e):
    EB, B, _ = e.shape

    def body(er, mr):
        i = pl.program_id(0)
        bm = jnp.max(er[...])

        @pl.when(i == 0)
        def _():
            mr[0, 0] = bm

        @pl.when(i > 0)
        def _():
            mr[0, 0] = jnp.maximum(mr[0, 0], bm)

    return pl.pallas_call(
        body,
        grid=(EB,),
        in_specs=[pl.BlockSpec((1, B, 1), lambda i: (i, 0, 0))],
        out_specs=_smem_full((1, 1)),
        out_shape=jax.ShapeDtypeStruct((1, 1), jnp.float32),
    )(e)


def _expshift(e, m):
    EB, B, _ = e.shape

    def body(er, mr, wr):
        wr[...] = jnp.exp(er[...] - mr[0, 0])

    return pl.pallas_call(
        body,
        grid=(EB,),
        in_specs=[pl.BlockSpec((1, B, 1), lambda i: (i, 0, 0)),
                  _smem_full((1, 1))],
        out_specs=pl.BlockSpec((1, B, 1), lambda i: (i, 0, 0)),
        out_shape=jax.ShapeDtypeStruct((EB, B, 1), jnp.float32),
    )(e, m)


def _denom(wb, dstb, NP):
    EB, _, B = wb.shape

    def body(wr, dr, denr):
        i = pl.program_id(0)

        @pl.when(i == 0)
        def _():
            def z(jb, _):
                for u in range(8):
                    denr[0, jb * 8 + u] = 0.0
                return 0
            lax.fori_loop(0, NP // 8, z, 0)

        def lp(jb, _):
            for u in range(16):
                j = jb * 16 + u
                d = dr[0, 0, j]
                denr[0, d] = denr[0, d] + wr[0, 0, j]
            return 0

        lax.fori_loop(0, B // 16, lp, 0)

    return pl.pallas_call(
        body,
        grid=(EB,),
        in_specs=[_smem_blk(B), _smem_blk(B)],
        out_specs=_smem_full((1, NP)),
        out_shape=jax.ShapeDtypeStruct((1, NP), jnp.float32),
    )(wb, dstb)


def _aggregate(srcb, dstb, wb, den, xl, bg_row, n):
    EB, _, B = wb.shape
    NP, H = xl.shape
    G = 32

    def body(sr, dr, wr, denr, xlr, bgr, outr):
        i = pl.program_id(0)

        @pl.when(i == 0)
        def _():
            outr[...] = jnp.zeros((NP, H), jnp.float32)

        def lp(jb, _):
            j0 = jb * G
            rows = [xlr[pl.ds(sr[0, 0, j0 + u], 1), :] for u in range(G)]
            for u in range(G):
                j = j0 + u
                d = dr[0, 0, j]
                alpha = wr[0, 0, j] / (denr[0, d] + 1e-16)
                outr[pl.ds(d, 1), :] = outr[pl.ds(d, 1), :] + alpha * rows[u]
            return 0

        lax.fori_loop(0, B // G, lp, 0)

        @pl.when(i == EB - 1)
        def _():
            acc = outr[...] + bgr[...]
            acc = jnp.where(acc >= 0, acc, 0.0)
            rid = lax.broadcasted_iota(jnp.int32, (NP, H), 0)
            outr[...] = jnp.where(rid < n, acc, 0.0)

    return pl.pallas_call(
        body,
        grid=(EB,),
        in_specs=[_smem_blk(B), _smem_blk(B), _smem_blk(B), _smem_full((1, NP)),
                  _full((NP, H)), _full((1, H))],
        out_specs=_full((NP, H)),
        out_shape=jax.ShapeDtypeStruct((NP, H), jnp.float32),
    )(srcb, dstb, wb, den, xl, bg_row)


def _tagg(srcb, dstb, t_row):
    EB, _, B = srcb.shape
    _, NP = t_row.shape

    def body(sr, dr, tr, ar):
        i = pl.program_id(0)

        @pl.when(i == 0)
        def _():
            def z(jb, _):
                for u in range(8):
                    ar[0, jb * 8 + u] = 0.0
                return 0
            lax.fori_loop(0, NP // 8, z, 0)

        def lp(jb, _):
            for u in range(16):
                j = jb * 16 + u
                s = sr[0, 0, j]
                d = dr[0, 0, j]
                ar[0, d] = ar[0, d] + tr[0, s]
            return 0

        lax.fori_loop(0, B // 16, lp, 0)

    return pl.pallas_call(
        body,
        grid=(EB,),
        in_specs=[_smem_blk(B), _smem_blk(B), _smem_full((1, NP))],
        out_specs=_smem_full((1, NP)),
        out_shape=jax.ShapeDtypeStruct((1, NP), jnp.float32),
    )(srcb, dstb, t_row)


def _scorevec(agg_row, root_row, pb, n):
    _, NP = agg_row.shape

    def body(ar, rr, pr, scr, thr):
        sc = ar[...] + rr[...] + pr[0, 0]
        lane = lax.broadcasted_iota(jnp.int32, (1, NP), 1)
        sc = jnp.where(lane < n, sc, -1e30)
        scr[...] = sc
        thr[...] = jnp.tanh(sc)

    return pl.pallas_call(
        body,
        in_specs=[_full0((1, NP)), _full0((1, NP)), _smem_full0((1, 1))],
        out_specs=[_full0((1, NP)), _full0((1, NP))],
        out_shape=[jax.ShapeDtypeStruct((1, NP), jnp.float32),
                   jax.ShapeDtypeStruct((1, NP), jnp.float32)],
    )(agg_row, root_row, pb)


def _rank_select(score_col, score_row, k):
    NP = score_row.shape[1]
    R = 8

    def body(cr, rr, selr):
        i = pl.program_id(0)
        sc_i = cr[...]
        sc_j = rr[...]
        ig = i * R + lax.broadcasted_iota(jnp.int32, (R, 1), 0)
        jg = lax.broadcasted_iota(jnp.int32, (R, NP), 1)
        gt = (sc_j > sc_i).astype(jnp.int32)
        tie = ((sc_j == sc_i) & (jg < ig)).astype(jnp.int32)
        rank = jnp.sum(gt + tie, axis=1, keepdims=True)
        selr[...] = (rank < k).astype(jnp.int32)

    return pl.pallas_call(
        body,
        grid=(NP // R,),
        in_specs=[pl.BlockSpec((R, 1), lambda i: (i, 0)), _full((1, NP))],
        out_specs=pl.BlockSpec((R, 1), lambda i: (i, 0)),
        out_shape=jax.ShapeDtypeStruct((NP, 1), jnp.int32),
    )(score_col, score_row)


def _prefix_map(sel_col, sel_row, k):
    NP = sel_row.shape[1]
    R = 8

    def body(cr, rr, mr):
        i = pl.program_id(0)
        ig = i * R + lax.broadcasted_iota(jnp.int32, (R, 1), 0)
        jg = lax.broadcasted_iota(jnp.int32, (R, NP), 1)
        cnt = jnp.sum(rr[...] * (jg < ig).astype(jnp.int32), axis=1, keepdims=True)
        mr[...] = jnp.where(cr[...] > 0, cnt, k)

    return pl.pallas_call(
        body,
        grid=(NP // R,),
        in_specs=[pl.BlockSpec((R, 1), lambda i: (i, 0)), _full((1, NP))],
        out_specs=pl.BlockSpec((R, 1), lambda i: (i, 0)),
        out_shape=jax.ShapeDtypeStruct((NP, 1), jnp.int32),
    )(sel_col, sel_row)


def _permbuild(map_row, k, KP):
    NP = map_row.shape[1]

    def body(mr, prf):
        def lp(ib, _):
            for u in range(4):
                i = ib * 4 + u
                m = mr[0, i]
                prf[0, jnp.minimum(m, k)] = i
            return 0

        lax.fori_loop(0, NP // 4, lp, 0)

    return pl.pallas_call(
        body,
        in_specs=[_smem_full0((1, NP))],
        out_specs=_smem_full0((1, KP)),
        out_shape=jax.ShapeDtypeStruct((1, KP), jnp.int32),
    )(map_row)


def _remap(srcb, dstb, map_row, k):
    EB, _, B = srcb.shape
    NP = map_row.shape[1]

    def body(sr, dr, mr, nsr, ndr):
        def lp(jb, _):
            for u in range(16):
                j = jb * 16 + u
                a = mr[0, sr[0, 0, j]]
                b = mr[0, dr[0, 0, j]]
                inv = (a == k) | (b == k)
                nsr[0, 0, j] = jnp.where(inv, k, a)
                ndr[0, 0, j] = jnp.where(inv, k, b)
            return 0

        lax.fori_loop(0, B // 16, lp, 0)

    return pl.pallas_call(
        body,
        grid=(EB,),
        in_specs=[_smem_blk(B), _smem_blk(B), _smem_full((1, NP))],
        out_specs=[_smem_blk(B), _smem_blk(B)],
        out_shape=[jax.ShapeDtypeStruct((EB, 1, B), jnp.int32),
                   jax.ShapeDtypeStruct((EB, 1, B), jnp.int32)],
    )(srcb, dstb, map_row)


def _pool(perm, th_row, x, k, NPn):
    NP, H = x.shape
    KP = perm.shape[1]

    def body(pr, thr, xr, xnr, ror):
        xnr[...] = jnp.zeros((NPn, H), jnp.float32)
        ror[...] = jnp.zeros((8, H), jnp.float32)

        def lp(j, carry):
            mx, sm = carry
            i = pr[0, j]
            r = xr[pl.ds(i, 1), :] * thr[0, i]
            xnr[pl.ds(j, 1), :] = r
            return jnp.maximum(mx, r), sm + r

        init = (jnp.full((1, H), -1e30, jnp.float32), jnp.zeros((1, H), jnp.float32))
        mx, sm = lax.fori_loop(0, k, lp, init)
        ror[pl.ds(0, 1), :] = mx
        ror[pl.ds(1, 1), :] = sm / k

    return pl.pallas_call(
        body,
        in_specs=[_smem_full0((1, KP)), _smem_full0((1, NP)), _full0((NP, H))],
        out_specs=[_full0((NPn, H)), _full0((8, H))],
        out_shape=[jax.ShapeDtypeStruct((NPn, H), jnp.float32),
                   jax.ShapeDtypeStruct((8, H), jnp.float32)],
    )(perm, th_row, x)


def _mlp(gs, L1w, L1b, L2w, L2b, L3wp, L3bp):
    def body(g1, g2, g3, g4, w1, b1, w2, b2, w3, b3, lgr, prr):
        g = g1[...] + g2[...] + g3[...] + g4[...]
        h1 = jnp.dot(g, w1[...], preferred_element_type=jnp.float32) + b1[...]
        h1 = jnp.where(h1 >= 0, h1, 0.0)
        h2 = jnp.dot(h1, w2[...], preferred_element_type=jnp.float32) + b2[...]
        h2 = jnp.where(h2 >= 0, h2, 0.0)
        lg = jnp.dot(h2, w3[...], preferred_element_type=jnp.float32) + b3[...]
        lane = lax.broadcasted_iota(jnp.int32, lg.shape, 1)
        valid = lane < 2
        lgm = jnp.where(valid, lg, -jnp.inf)
        z = lgm - jnp.max(lgm)
        ez = jnp.where(valid, jnp.exp(z), 0.0)
        prr[...] = ez / jnp.sum(ez)
        lgr[...] = lg

    n_in = 7
    return pl.pallas_call(
        body,
        in_specs=[_full0(a.shape) for a in gs] +
                 [_full0(L1w.shape), _full0(L1b.shape), _full0(L2w.shape),
                  _full0(L2b.shape), _full0(L3wp.shape), _full0(L3bp.shape)],
        out_specs=[_full0((1, 128)), _full0((1, 128))],
        out_shape=[jax.ShapeDtypeStruct((1, 128), jnp.float32),
                   jax.ShapeDtypeStruct((1, 128), jnp.float32)],
    )(*gs, L1w, L1b, L2w, L2b, L3wp, L3bp)


def _blocks(idx, fill, B):
    n = idx.shape[0]
    EB = _ru(n, B) // B
    pad = jnp.full((EB * B - n,), fill, jnp.int32)
    return jnp.concatenate([idx.astype(jnp.int32), pad]).reshape(EB, 1, B)


@jax.jit
def kernel(x, edge_index, batch, W1l, W1r, a1, bg1, W2l, W2r, a2, bg2, W3l, W3r,
           a3, bg3, W4l, W4r, a4, bg4, P1root, P1rel, P1b, P2root, P2rel, P2b,
           P3root, P3rel, P3b, P4root, P4rel, P4b, L1w, L1b, L2w, L2b, L3w, L3b):
    N0, DIN = x.shape
    E = edge_index.shape[1]
    H = W1l.shape[1]

    gat_w = [(W1l, W1r, a1, bg1), (W2l, W2r, a2, bg2), (W3l, W3r, a3, bg3),
             (W4l, W4r, a4, bg4)]
    pool_w = [(P1root, P1rel, P1b), (P2root, P2rel, P2b), (P3root, P3rel, P3b),
              (P4root, P4rel, P4b)]

    src = edge_index[0].astype(jnp.int32)
    dst = edge_index[1].astype(jnp.int32)
    n = N0
    NP = _ru(n + 1, 128)
    Xp = jnp.zeros((NP, DIN), jnp.float32).at[:n].set(x)

    readouts = []
    for li in range(4):
        Wl, Wr, att, bg = gat_w[li]
        Proot, Prel, Pb = pool_w[li]
        k = n // 2

        loops = jnp.arange(n, dtype=jnp.int32)
        s_full = jnp.concatenate([src, loops])
        d_full = jnp.concatenate([dst, loops])
        sb = _blocks(s_full, n, _B)
        db = _blocks(d_full, n, _B)

        xl = _mm(Xp, Wl)
        xr = _mm(Xp, Wr)
        att_row = att.reshape(1, H)

        e = _edge_logits(sb, db, xl, xr, att_row)
        m = _gmax(e)
        w = _expshift(e, m)
        EBn = e.shape[0]
        w13 = w.reshape(EBn, 1, _B)
        den = _denom(w13, db, NP)
        xg = _aggregate(sb, db, w13, den, xl, bg.reshape(1, H), n)

        W2c = jnp.zeros((H, 128), jnp.float32)
        W2c = W2c.at[:, 0].set(Prel[:, 0]).at[:, 1].set(Proot[:, 0])
        tc = _mm(xg, W2c)
        t_row = tc[:, 0:1].reshape(1, NP)
        root_row = tc[:, 1:2].reshape(1, NP)

        sbo = _blocks(src, n, _B)
        dbo = _blocks(dst, n, _B)
        agg_row = _tagg(sbo, dbo, t_row)
        score, th = _scorevec(agg_row, root_row, Pb.reshape(1, 1), n)

        score_col = score.reshape(NP, 1)
        sel = _rank_select(score_col, score, k)
        mapping = _prefix_map(sel, sel.reshape(1, NP), k)
        map_row = mapping.reshape(1, NP)

        KP = _ru(k + 1, 128)
        perm = _permbuild(map_row, k, KP)

        nsb, ndb = _remap(sbo, dbo, map_row, k)
        src = nsb.reshape(-1)[:E]
        dst = ndb.reshape(-1)[:E]

        NPn = _ru(k + 1, 128)
        Xp, ro = _pool(perm, th, xg, k, NPn)
        readouts.append(ro[0:2, :].reshape(1, 2 * H))
        n = k
        NP = NPn

    lgp, prp = _mlp(readouts, L1w, L1b.reshape(1, H),
                    L2w, L2b.reshape(1, H // 2),
                    jnp.pad(L3w, ((0, 0), (0, 126))),
                    jnp.pad(L3b.reshape(1, 2), ((0, 0), (0, 126))))
    return lgp[:, :2], prp[:, :2]
